# baseline (speedup 1.0000x reference)
import functools

import jax
import jax.numpy as jnp
from jax import lax
from jax.experimental import pallas as pl
from jax.experimental.pallas import tpu as pltpu
from jax.experimental.pallas import tpu_sc as plsc

LEAKY = 0.2
GATHER_WINDOW = 128
ROW_BLOCK = 2048


def _gather_edge_rows(x, idx_pad):
    P = idx_pad.shape[1]
    D = x.shape[1]
    mesh = plsc.VectorSubcoreMesh(core_axis_name="core",
                                  subcore_axis_name="subcore")
    row_ty = jax.ShapeDtypeStruct((P, D), x.dtype)

    @functools.partial(pl.kernel, out_type=(row_ty, row_ty), mesh=mesh)
    def k(x_hbm, i_hbm, src_hbm, dst_hbm):
        def body(si_vmem, di_vmem, so_vmem, do_vmem):
            def go(sem_a, sem_b):
                a = pltpu.async_copy(x_hbm.at[si_vmem.at[0]], so_vmem, sem_a)
                b = pltpu.async_copy(x_hbm.at[di_vmem.at[0]], do_vmem, sem_b)
                a.wait()
                b.wait()

            pl.run_scoped(go, pltpu.SemaphoreType.DMA, pltpu.SemaphoreType.DMA)

        pltpu.emit_pipeline(
            body,
            grid=(P // GATHER_WINDOW,),
            in_specs=[
                pl.BlockSpec((1, GATHER_WINDOW), lambda i: (0, i)),
                pl.BlockSpec((1, GATHER_WINDOW), lambda i: (1, i)),
            ],
            out_specs=[
                pl.BlockSpec((GATHER_WINDOW, D), lambda i: (i, 0)),
                pl.BlockSpec((GATHER_WINDOW, D), lambda i: (i, 0)),
            ],
            core_axis_name=("core", "subcore"),
            dimension_semantics=(pltpu.PARALLEL,),
        )(i_hbm, i_hbm, src_hbm, dst_hbm)

    return k(x, idx_pad)


def _sort_lanes(v):
    n = v.shape[-1]
    lane = lax.broadcasted_iota(jnp.int32, v.shape, len(v.shape) - 1)
    k = 2
    while k <= n:
        s = k // 2
        while s >= 1:
            lower = (lane & s) == 0
            vr = pltpu.roll(v, n - s, 1)
            vl = pltpu.roll(v, s, 1)
            partner = jnp.where(lower, vr, vl)
            keepmin = ((lane & k) == 0) == lower
            v = jnp.where(keepmin, jnp.minimum(v, partner),
                          jnp.maximum(v, partner))
            s //= 2
        k *= 2
    return v


def _tc_body(_, x_ref, s_ref, d_ref, wi_ref, wh_ref, o_ref, x1_scr):
    f32 = jnp.float32
    wi = wi_ref[...]
    x1_scr[...] = jnp.dot(x_ref[...], wi, preferred_element_type=f32)
    diff = (s_ref[...] - d_ref[...]).astype(jnp.bfloat16)
    v = jnp.dot(diff, wi, preferred_element_type=f32).astype(jnp.bfloat16)
    v = _sort_lanes(v)
    ones = jnp.ones((wh_ref.shape[1], 1), f32)
    wsum = jnp.dot(wh_ref[...], ones, preferred_element_type=f32)
    s = jnp.dot(v.astype(f32), wsum, preferred_element_type=f32)
    x1 = x1_scr[...]
    o_ref[...] = (1.0 - LEAKY) * x1 + LEAKY * jnp.tanh(x1 + s)


N_CHUNKS = 4


def kernel(x, edge_index, wi, wh):
    N, D = x.shape
    grid_n = pl.cdiv(N, ROW_BLOCK)
    blocks_per_chunk = pl.cdiv(grid_n, N_CHUNKS)
    chunk_rows = blocks_per_chunk * ROW_BLOCK
    P = N_CHUNKS * chunk_rows
    assert chunk_rows % GATHER_WINDOW == 0
    idx_pad = jnp.pad(edge_index, ((0, 0), (0, P - N)))

    gathered = []
    for c in range(N_CHUNKS):
        idx_c = lax.dynamic_slice_in_dim(idx_pad, c * chunk_rows, chunk_rows,
                                         axis=1)
        gathered.append(_gather_edge_rows(x, idx_c))

    zero = lambda i: (0, 0)
    out = jnp.zeros_like(x)
    for c in range(N_CHUNKS):
        src_rows, dst_rows = gathered[c]
        n_blk = min(blocks_per_chunk, grid_n - c * blocks_per_chunk)
        off = c * blocks_per_chunk
        blk_o = lambda i, off=off: (i + off, 0)
        blk_c = lambda i: (i, 0)
        out = pl.pallas_call(
            _tc_body,
            grid=(n_blk,),
            in_specs=[
                pl.BlockSpec(memory_space=pl.ANY),
                pl.BlockSpec((ROW_BLOCK, D), blk_o),
                pl.BlockSpec((ROW_BLOCK, D), blk_c),
                pl.BlockSpec((ROW_BLOCK, D), blk_c),
                pl.BlockSpec((D, D), zero),
                pl.BlockSpec((D, D), zero),
            ],
            out_specs=pl.BlockSpec((ROW_BLOCK, D), blk_o),
            out_shape=jax.ShapeDtypeStruct((N, D), x.dtype),
            scratch_shapes=[
                pltpu.VMEM((ROW_BLOCK, D), jnp.float32),
            ],
            input_output_aliases={0: 0},
        )(out, x, src_rows, dst_rows, wi, wh)
    return out


# speedup vs baseline: 1.5157x; 1.5157x over previous
import functools

import jax
import jax.numpy as jnp
from jax import lax
from jax.experimental import pallas as pl
from jax.experimental.pallas import tpu as pltpu
from jax.experimental.pallas import tpu_sc as plsc

LEAKY = 0.2
GATHER_WINDOW = 128
ROW_BLOCK = 1024


def _gather_edge_rows(x, idx_pad):
    P = idx_pad.shape[1]
    D = x.shape[1]
    mesh = plsc.VectorSubcoreMesh(core_axis_name="core",
                                  subcore_axis_name="subcore")
    row_ty = jax.ShapeDtypeStruct((P, D), x.dtype)

    @functools.partial(pl.kernel, out_type=(row_ty, row_ty), mesh=mesh)
    def k(x_hbm, i_hbm, src_hbm, dst_hbm):
        def body(si_vmem, di_vmem, so_vmem, do_vmem):
            def go(sem_a, sem_b):
                a = pltpu.async_copy(x_hbm.at[si_vmem.at[0]], so_vmem, sem_a)
                b = pltpu.async_copy(x_hbm.at[di_vmem.at[0]], do_vmem, sem_b)
                a.wait()
                b.wait()

            pl.run_scoped(go, pltpu.SemaphoreType.DMA, pltpu.SemaphoreType.DMA)

        pltpu.emit_pipeline(
            body,
            grid=(P // GATHER_WINDOW,),
            in_specs=[
                pl.BlockSpec((1, GATHER_WINDOW), lambda i: (0, i)),
                pl.BlockSpec((1, GATHER_WINDOW), lambda i: (1, i)),
            ],
            out_specs=[
                pl.BlockSpec((GATHER_WINDOW, D), lambda i: (i, 0)),
                pl.BlockSpec((GATHER_WINDOW, D), lambda i: (i, 0)),
            ],
            core_axis_name=("core", "subcore"),
            dimension_semantics=(pltpu.PARALLEL,),
        )(i_hbm, i_hbm, src_hbm, dst_hbm)

    return k(x, idx_pad)


def _sort_lanes(v):
    n = v.shape[-1]
    lane = lax.broadcasted_iota(jnp.int32, v.shape, len(v.shape) - 1)
    k = 2
    while k <= n:
        s = k // 2
        while s >= 1:
            lower = (lane & s) == 0
            vr = pltpu.roll(v, n - s, 1)
            vl = pltpu.roll(v, s, 1)
            partner = jnp.where(lower, vr, vl)
            keepmin = ((lane & k) == 0) == lower
            v = jnp.where(keepmin, jnp.minimum(v, partner),
                          jnp.maximum(v, partner))
            s //= 2
        k *= 2
    return v


def _tc_body(_, x_ref, s_ref, d_ref, wi_ref, wh_ref, o_ref, x1_scr):
    f32 = jnp.float32
    wi = wi_ref[...]
    x1_scr[...] = jnp.dot(x_ref[...], wi, preferred_element_type=f32)
    diff = (s_ref[...] - d_ref[...]).astype(jnp.bfloat16)
    v = jnp.dot(diff, wi, preferred_element_type=f32).astype(jnp.bfloat16)
    v = _sort_lanes(v)
    ones = jnp.ones((wh_ref.shape[1], 1), f32)
    wsum = jnp.dot(wh_ref[...], ones, preferred_element_type=f32)
    s = jnp.dot(v.astype(f32), wsum, preferred_element_type=f32)
    x1 = x1_scr[...]
    o_ref[...] = (1.0 - LEAKY) * x1 + LEAKY * jnp.tanh(x1 + s)


N_CHUNKS = 4


def kernel(x, edge_index, wi, wh):
    N, D = x.shape
    grid_n = pl.cdiv(N, ROW_BLOCK)
    blocks_per_chunk = pl.cdiv(grid_n, N_CHUNKS)
    chunk_rows = blocks_per_chunk * ROW_BLOCK
    P = N_CHUNKS * chunk_rows
    assert chunk_rows % GATHER_WINDOW == 0
    idx_pad = jnp.pad(edge_index, ((0, 0), (0, P - N)))

    gathered = []
    for c in range(N_CHUNKS):
        idx_c = lax.dynamic_slice_in_dim(idx_pad, c * chunk_rows, chunk_rows,
                                         axis=1)
        gathered.append(_gather_edge_rows(x, idx_c))

    zero = lambda i: (0, 0)
    out = jnp.zeros_like(x)
    for c in range(N_CHUNKS):
        src_rows, dst_rows = gathered[c]
        n_blk = min(blocks_per_chunk, grid_n - c * blocks_per_chunk)
        off = c * blocks_per_chunk
        blk_o = lambda i, off=off: (i + off, 0)
        blk_c = lambda i: (i, 0)
        out = pl.pallas_call(
            _tc_body,
            grid=(n_blk,),
            in_specs=[
                pl.BlockSpec(memory_space=pl.ANY),
                pl.BlockSpec((ROW_BLOCK, D), blk_o),
                pl.BlockSpec((ROW_BLOCK, D), blk_c),
                pl.BlockSpec((ROW_BLOCK, D), blk_c),
                pl.BlockSpec((D, D), zero),
                pl.BlockSpec((D, D), zero),
            ],
            out_specs=pl.BlockSpec((ROW_BLOCK, D), blk_o),
            out_shape=jax.ShapeDtypeStruct((N, D), x.dtype),
            scratch_shapes=[
                pltpu.VMEM((ROW_BLOCK, D), jnp.float32),
            ],
            input_output_aliases={0: 0},
        )(out, x, src_rows, dst_rows, wi, wh)
    return out
